# Initial kernel scaffold; baseline (speedup 1.0000x reference)
import jax
import jax.numpy as jnp
from jax.experimental import pallas as pl


def kernel(x, edge_index_e0, edge_index_e1, edge_index_e2, Ws, bs):
    raise NotImplementedError("write your pallas kernel here")


# speedup vs baseline: 2.7871x; 2.7871x over previous
import functools

import jax
import jax.numpy as jnp
from jax import lax
from jax.experimental import pallas as pl
from jax.experimental.pallas import tpu as pltpu
from jax.experimental.pallas import tpu_sc as plsc

N = 10000
E = 320000
D_IN = 128
D_H = 128
D_OUT = 64

NC = 2
NS = 16
NW = NC * NS

K = 128
NB = 79
EPW = NB * K
E_PAD = NW * EPW
SUBROWS = 640
N_PAD = NS * SUBROWS

_MESH = plsc.VectorSubcoreMesh(core_axis_name="c", subcore_axis_name="s")


DEGW = 16


@functools.partial(
    pl.kernel,
    out_type=jax.ShapeDtypeStruct((2, 6, N_PAD, DEGW), jnp.float32),
    mesh=_MESH,
    scratch_types=[
        pltpu.VMEM((K,), jnp.int32),
        pltpu.VMEM((K, DEGW), jnp.float32),
        pltpu.VMEM((SUBROWS, DEGW), jnp.float32),
        pltpu.VMEM_SHARED((N_PAD, DEGW), jnp.float32),
    ],
    compiler_params=pltpu.CompilerParams(use_tc_tiling_on_sc=False),
)
def _degrees(i0, i1, i2, i3, i4, i5, ones_hbm, zeros_hbm, out,
             idx_v, ones_v, zb_v, acc):
    c = lax.axis_index("c")
    s = lax.axis_index("s")
    base = (s * NC + c) * EPW
    row0 = s * SUBROWS

    pltpu.sync_copy(ones_hbm, ones_v)
    pltpu.sync_copy(zeros_hbm, zb_v)

    for r, ref in enumerate((i0, i1, i2, i3, i4, i5)):
        pltpu.sync_copy(zb_v, acc.at[pl.ds(row0, SUBROWS)])
        plsc.subcore_barrier()

        def _acc(b, _, ref=ref):
            pltpu.sync_copy(ref.at[pl.ds(base + b * K, K)], idx_v)
            pltpu.sync_copy(ones_v, acc.at[idx_v], add=True)
            return 0
        lax.fori_loop(0, NB, _acc, 0)
        plsc.subcore_barrier()

        pltpu.sync_copy(acc.at[pl.ds(row0, SUBROWS)],
                        out.at[c, r, pl.ds(row0, SUBROWS)])
        plsc.subcore_barrier()


def _make_propagate(d):
    @functools.partial(
        pl.kernel,
        out_type=jax.ShapeDtypeStruct((3, 2, N_PAD, d), jnp.float32),
        mesh=_MESH,
        scratch_types=[
            pltpu.VMEM((K,), jnp.int32),
            pltpu.VMEM((K,), jnp.int32),
            pltpu.VMEM((K, d), jnp.float32),
            pltpu.VMEM((64, d), jnp.float32),
            pltpu.VMEM_SHARED((N_PAD, d), jnp.float32),
            pltpu.SemaphoreType.DMA,
        ],
        compiler_params=pltpu.CompilerParams(use_tc_tiling_on_sc=False),
    )
    def _propagate(m0, m1, m2, s0, d0, s1, d1, s2, d2, zeros_hbm, out,
                   srcv, dstv, rows, zrows, acc, sem):
        c = lax.axis_index("c")
        s = lax.axis_index("s")
        base = (s * NC + c) * EPW
        row0 = s * SUBROWS

        pltpu.sync_copy(zeros_hbm, zrows)

        for e, (msrc, esrc, edst) in enumerate(
                ((m0, s0, d0), (m1, s1, d1), (m2, s2, d2))):
            for j in range(SUBROWS // 64):
                pltpu.sync_copy(zrows, acc.at[pl.ds(row0 + j * 64, 64)])
            plsc.subcore_barrier()

            def _step(b, _, msrc=msrc, esrc=esrc, edst=edst):
                off = base + b * K
                pltpu.sync_copy(esrc.at[pl.ds(off, K)], srcv)
                pltpu.sync_copy(edst.at[pl.ds(off, K)], dstv)
                pltpu.async_copy(msrc.at[srcv], rows, sem).wait()
                pltpu.sync_copy(rows, acc.at[dstv], add=True)
                return 0
            lax.fori_loop(0, NB, _step, 0)
            plsc.subcore_barrier()

            pltpu.sync_copy(acc.at[pl.ds(row0, SUBROWS)],
                            out.at[e, c, pl.ds(row0, SUBROWS)])
            plsc.subcore_barrier()

    return _propagate


_propagate128 = _make_propagate(D_H)
_propagate64 = _make_propagate(D_OUT)


_R = 1024


def _scales(degs, e, which):
    cnt = degs[0, 2 * e + which, :, 0] + degs[1, 2 * e + which, :, 0]
    return lax.rsqrt(jnp.maximum(cnt, 1.0))


def _t0_body(x_ref, degs_ref, w_ref, o0, o1, o2):
    x = x_ref[...]
    degs = degs_ref[...]
    w = w_ref[...]
    outs = (o0, o1, o2)
    for e in range(3):
        xs = x * _scales(degs, e, 0)[:, None]
        outs[e][...] = jnp.dot(xs, w[e], preferred_element_type=jnp.float32)


def _mk_t0():
    grid = (N_PAD // _R,)
    return pl.pallas_call(
        _t0_body,
        grid=grid,
        in_specs=[
            pl.BlockSpec((_R, D_IN), lambda i: (i, 0)),
            pl.BlockSpec((2, 6, _R, DEGW), lambda i: (0, 0, i, 0)),
            pl.BlockSpec((3, D_IN, D_H), lambda i: (0, 0, 0)),
        ],
        out_specs=[
            pl.BlockSpec((_R, D_H), lambda i: (i, 0)),
            pl.BlockSpec((_R, D_H), lambda i: (i, 0)),
            pl.BlockSpec((_R, D_H), lambda i: (i, 0)),
        ],
        out_shape=[jax.ShapeDtypeStruct((N_PAD, D_H), jnp.float32)] * 3,
    )


def _mid_body(part_ref, degs_ref, w_ref, b_ref, o0, o1, o2, *, dp, do):
    part = part_ref[...]
    degs = degs_ref[...]
    w = w_ref[...]
    h = jnp.zeros((_R, dp), jnp.float32)
    for e in range(3):
        h = h + (part[e, 0] + part[e, 1]) * _scales(degs, e, 1)[:, None]
    h = jnp.maximum(h + b_ref[...][0], 0.0)
    outs = (o0, o1, o2)
    for e in range(3):
        hs = h * _scales(degs, e, 0)[:, None]
        outs[e][...] = jnp.dot(hs, w[e], preferred_element_type=jnp.float32)


def _mk_mid(dp, do):
    return pl.pallas_call(
        functools.partial(_mid_body, dp=dp, do=do),
        grid=(N_PAD // _R,),
        in_specs=[
            pl.BlockSpec((3, 2, _R, dp), lambda i: (0, 0, i, 0)),
            pl.BlockSpec((2, 6, _R, DEGW), lambda i: (0, 0, i, 0)),
            pl.BlockSpec((3, dp, do), lambda i: (0, 0, 0)),
            pl.BlockSpec((1, dp), lambda i: (0, 0)),
        ],
        out_specs=[
            pl.BlockSpec((_R, do), lambda i: (i, 0)),
            pl.BlockSpec((_R, do), lambda i: (i, 0)),
            pl.BlockSpec((_R, do), lambda i: (i, 0)),
        ],
        out_shape=[jax.ShapeDtypeStruct((N_PAD, do), jnp.float32)] * 3,
    )


def _final_body(part_ref, degs_ref, b_ref, o_ref):
    part = part_ref[...]
    degs = degs_ref[...]
    h = jnp.zeros((_R, D_OUT), jnp.float32)
    for e in range(3):
        h = h + (part[e, 0] + part[e, 1]) * _scales(degs, e, 1)[:, None]
    o_ref[...] = h + b_ref[...][0]


def _mk_final():
    return pl.pallas_call(
        _final_body,
        grid=(N_PAD // _R,),
        in_specs=[
            pl.BlockSpec((3, 2, _R, D_OUT), lambda i: (0, 0, i, 0)),
            pl.BlockSpec((2, 6, _R, DEGW), lambda i: (0, 0, i, 0)),
            pl.BlockSpec((1, D_OUT), lambda i: (0, 0)),
        ],
        out_specs=pl.BlockSpec((_R, D_OUT), lambda i: (i, 0)),
        out_shape=jax.ShapeDtypeStruct((N_PAD, D_OUT), jnp.float32),
    )


def kernel(x, edge_index_e0, edge_index_e1, edge_index_e2, Ws, bs):
    pad = jnp.full((E_PAD - E,), N, jnp.int32)
    idx = []
    for ei in (edge_index_e0, edge_index_e1, edge_index_e2):
        idx.append(jnp.concatenate([ei[0], pad]))
        idx.append(jnp.concatenate([ei[1], pad]))

    ones = jnp.ones((K, DEGW), jnp.float32)
    zdeg = jnp.zeros((SUBROWS, DEGW), jnp.float32)
    z128 = jnp.zeros((64, D_H), jnp.float32)
    z64 = jnp.zeros((64, D_OUT), jnp.float32)

    degs = _degrees(*idx, ones, zdeg)

    x_pad = jnp.zeros((N_PAD, D_IN), x.dtype).at[:N].set(x)
    w = [jnp.stack(Ws[l]) for l in range(4)]
    bvec = [(bs[l][0] + bs[l][1] + bs[l][2])[None, :] for l in range(4)]

    srcs = (idx[0], idx[2], idx[4])
    dsts = (idx[1], idx[3], idx[5])

    def prop(fn, m, z):
        return fn(m[0], m[1], m[2], srcs[0], dsts[0],
                  srcs[1], dsts[1], srcs[2], dsts[2], z)

    m = _mk_t0()(x_pad, degs, w[0])
    part = prop(_propagate128, m, z128)
    m = _mk_mid(D_H, D_H)(part, degs, w[1], bvec[0])
    part = prop(_propagate128, m, z128)
    m = _mk_mid(D_H, D_H)(part, degs, w[2], bvec[1])
    part = prop(_propagate128, m, z128)
    m = _mk_mid(D_H, D_OUT)(part, degs, w[3], bvec[2])
    part = prop(_propagate64, m, z64)
    out = _mk_final()(part, degs, bvec[3])
    return out[:N]
